# initial kernel scaffold (unmeasured)
import jax
import jax.numpy as jnp
from jax import lax
from jax.experimental import pallas as pl
from jax.experimental.pallas import tpu as pltpu

N_DEV = 8
LOG2_N = 3
B, SQ, SKV, HQ_SH, DH = 2, 128, 128, 4, 64
D_MODEL = 512


def kernel(x, Wq, K_ext, V_ext, Wo):
    p = lax.axis_index("i")
    K_sh = lax.dynamic_slice_in_dim(K_ext, p * HQ_SH, HQ_SH, axis=2)
    V_sh = lax.dynamic_slice_in_dim(V_ext, p * HQ_SH, HQ_SH, axis=2)

    def body(x_ref, wq_ref, k_ref, v_ref, wo_ref, out_ref,
             send_buf, recv_buf, send_sems, recv_sems):
        my = lax.axis_index("i")

        barrier = pltpu.get_barrier_semaphore()
        for d in range(LOG2_N):
            partner = jnp.bitwise_xor(my, 1 << d)
            pl.semaphore_signal(
                barrier, inc=1,
                device_id=(partner,), device_id_type=pl.DeviceIdType.MESH,
            )
        pl.semaphore_wait(barrier, LOG2_N)

        qb = lax.broadcasted_iota(jnp.int32, (SQ, SKV), 0) // 64
        kb = lax.broadcasted_iota(jnp.int32, (SQ, SKV), 1) // 64
        mask = (qb == kb) | (kb == 0) | ((qb + kb) % 3 == 0)

        wq = wq_ref[...].astype(jnp.bfloat16)
        wo = wo_ref[...].astype(jnp.bfloat16)
        for b in range(B):
            xb = x_ref[b].astype(jnp.bfloat16)
            q = jnp.dot(xb, wq, preferred_element_type=jnp.float32)
            ctxs = []
            for h in range(HQ_SH):
                qh = q[:, h * DH:(h + 1) * DH].astype(jnp.bfloat16)
                kh = k_ref[b, :, h, :].astype(jnp.bfloat16)
                vh = v_ref[b, :, h, :].astype(jnp.bfloat16)
                s = lax.dot_general(
                    qh, kh, (((1,), (1,)), ((), ())),
                    preferred_element_type=jnp.float32,
                ) * 0.125
                s = jnp.where(mask, s, -1e9)
                m = jnp.max(s, axis=-1, keepdims=True)
                w = jnp.exp(s - m)
                w = w / jnp.sum(w, axis=-1, keepdims=True)
                ctxs.append(jnp.dot(w.astype(jnp.bfloat16), vh,
                                    preferred_element_type=jnp.float32))
            ctx = jnp.concatenate(ctxs, axis=1).astype(jnp.bfloat16)
            partial = jnp.dot(ctx, wo, preferred_element_type=jnp.float32)
            out_ref[b, :, :] = partial
            send_buf[b, :, :] = partial.astype(jnp.bfloat16)

        for d in range(LOG2_N):
            partner = jnp.bitwise_xor(my, 1 << d)
            rdma = pltpu.make_async_remote_copy(
                src_ref=send_buf,
                dst_ref=recv_buf.at[d],
                send_sem=send_sems.at[d],
                recv_sem=recv_sems.at[d],
                device_id=(partner,),
                device_id_type=pl.DeviceIdType.MESH,
            )
            rdma.start()
            rdma.wait()
            out_ref[...] += recv_buf[d].astype(jnp.float32)
            if d < LOG2_N - 1:
                send_buf[...] = out_ref[...].astype(jnp.bfloat16)

    return pl.pallas_call(
        body,
        out_shape=jax.ShapeDtypeStruct((B, SQ, D_MODEL), jnp.float32),
        in_specs=[pl.BlockSpec(memory_space=pltpu.VMEM)] * 5,
        out_specs=pl.BlockSpec(memory_space=pltpu.VMEM),
        scratch_shapes=[
            pltpu.VMEM((B, SQ, D_MODEL), jnp.bfloat16),
            pltpu.VMEM((LOG2_N, B, SQ, D_MODEL), jnp.bfloat16),
            pltpu.SemaphoreType.DMA((LOG2_N,)),
            pltpu.SemaphoreType.DMA((LOG2_N,)),
        ],
        compiler_params=pltpu.CompilerParams(collective_id=0),
    )(x, Wq, K_sh, V_sh, Wo)


# baseline (device time: 17655 ns/iter reference)
import jax
import jax.numpy as jnp
from jax import lax
from jax.experimental import pallas as pl
from jax.experimental.pallas import tpu as pltpu

N_DEV = 8
LOG2_N = 3
MASKS = (1, 3, 4)
B, SQ, SKV, HQ_SH, DH = 2, 128, 128, 4, 64
D_MODEL = 512
N_CHUNK = 4
D_HALF = D_MODEL // 2


def kernel(x, Wq, K_ext, V_ext, Wo):
    p = lax.axis_index("i")
    K_sh = lax.dynamic_slice_in_dim(K_ext, p * HQ_SH, HQ_SH, axis=2)
    V_sh = lax.dynamic_slice_in_dim(V_ext, p * HQ_SH, HQ_SH, axis=2)

    def body(x_ref, wq_ref, k_ref, v_ref, wo_ref, out_ref,
             send_buf, recv_buf, send_sems, recv_sems):
        my = lax.axis_index("i")
        partners = [jnp.bitwise_xor(my, m) for m in MASKS]

        barrier = pltpu.get_barrier_semaphore()
        for d in range(LOG2_N):
            pl.semaphore_signal(
                barrier, inc=1,
                device_id=(partners[d],), device_id_type=pl.DeviceIdType.MESH,
            )
        pl.semaphore_wait(barrier, LOG2_N)

        def make_rdma(d, c):
            return pltpu.make_async_remote_copy(
                src_ref=send_buf.at[d, c],
                dst_ref=recv_buf.at[d, c],
                send_sem=send_sems.at[d, c],
                recv_sem=recv_sems.at[d, c],
                device_id=(partners[d],),
                device_id_type=pl.DeviceIdType.MESH,
            )

        rdmas = [[make_rdma(d, c) for c in range(N_CHUNK)] for d in range(LOG2_N)]

        qb = lax.broadcasted_iota(jnp.int32, (SQ, SKV), 0) // 64
        kb = lax.broadcasted_iota(jnp.int32, (SQ, SKV), 1) // 64
        mask = (qb == kb) | (kb == 0) | ((qb + kb) % 3 == 0)
        bias = jnp.where(mask, 0.0, -1e9).astype(jnp.float32)

        wq = wq_ref[...].astype(jnp.bfloat16)
        wo = wo_ref[...].astype(jnp.bfloat16)
        for b in range(B):
            xb = x_ref[b].astype(jnp.bfloat16)
            q = jnp.dot(xb, wq, preferred_element_type=jnp.float32)
            ctxs = []
            for h in range(HQ_SH):
                qh = q[:, h * DH:(h + 1) * DH].astype(jnp.bfloat16)
                kh = k_ref[b, :, h, :].astype(jnp.bfloat16)
                vh = v_ref[b, :, h, :].astype(jnp.bfloat16)
                s = lax.dot_general(
                    qh, kh, (((1,), (1,)), ((), ())),
                    preferred_element_type=jnp.float32,
                ) * 0.125 + bias
                m = jnp.max(s, axis=-1, keepdims=True)
                w = jnp.exp(s - m)
                w = w / jnp.sum(w, axis=-1, keepdims=True)
                ctxs.append(jnp.dot(w.astype(jnp.bfloat16), vh,
                                    preferred_element_type=jnp.float32))
            ctx = jnp.concatenate(ctxs, axis=1).astype(jnp.bfloat16)
            partial = jnp.dot(ctx, wo, preferred_element_type=jnp.float32)
            out_ref[b, :, :] = partial
            for j in range(2):
                c = 2 * b + j
                send_buf[0, c, :, :] = (
                    partial[:, j * D_HALF:(j + 1) * D_HALF].astype(jnp.bfloat16)
                )
                rdmas[0][c].start()

        for d in range(LOG2_N):
            for c in range(N_CHUNK):
                b, j = c // 2, c % 2
                rdmas[d][c].wait_recv()
                acc = (out_ref[b, :, j * D_HALF:(j + 1) * D_HALF]
                       + recv_buf[d, c].astype(jnp.float32))
                out_ref[b, :, j * D_HALF:(j + 1) * D_HALF] = acc
                if d < LOG2_N - 1:
                    send_buf[d + 1, c, :, :] = acc.astype(jnp.bfloat16)
                    rdmas[d + 1][c].start()

        for d in range(LOG2_N):
            for c in range(N_CHUNK):
                rdmas[d][c].wait_send()

    return pl.pallas_call(
        body,
        out_shape=jax.ShapeDtypeStruct((B, SQ, D_MODEL), jnp.float32),
        in_specs=[pl.BlockSpec(memory_space=pltpu.VMEM)] * 5,
        out_specs=pl.BlockSpec(memory_space=pltpu.VMEM),
        scratch_shapes=[
            pltpu.VMEM((LOG2_N, N_CHUNK, SQ, D_HALF), jnp.bfloat16),
            pltpu.VMEM((LOG2_N, N_CHUNK, SQ, D_HALF), jnp.bfloat16),
            pltpu.SemaphoreType.DMA((LOG2_N, N_CHUNK)),
            pltpu.SemaphoreType.DMA((LOG2_N, N_CHUNK)),
        ],
        compiler_params=pltpu.CompilerParams(collective_id=0),
    )(x, Wq, K_sh, V_sh, Wo)


# device time: 16962 ns/iter; 1.0409x vs baseline; 1.0409x over previous
import jax
import jax.numpy as jnp
from jax import lax
from jax.experimental import pallas as pl
from jax.experimental.pallas import tpu as pltpu

N_DEV = 8
LOG2_N = 3
MASKS = (1, 3, 4)
B, SQ, SKV, HQ_SH, DH = 2, 128, 128, 4, 64
D_MODEL = 512
N_CHUNK = 4
D_HALF = D_MODEL // 2


def kernel(x, Wq, K_ext, V_ext, Wo):
    p = lax.axis_index("i")
    K_sh = lax.dynamic_slice_in_dim(K_ext, p * HQ_SH, HQ_SH, axis=2)
    V_sh = lax.dynamic_slice_in_dim(V_ext, p * HQ_SH, HQ_SH, axis=2)

    def body(x_ref, wq_ref, k_ref, v_ref, wo_ref, out_ref,
             send_buf, recv_buf, send_sems, recv_sems):
        my = lax.axis_index("i")
        partners = [jnp.bitwise_xor(my, m) for m in MASKS]

        barrier = pltpu.get_barrier_semaphore()
        for d in range(LOG2_N):
            pl.semaphore_signal(
                barrier, inc=1,
                device_id=(partners[d],), device_id_type=pl.DeviceIdType.MESH,
            )
        pl.semaphore_wait(barrier, LOG2_N)

        def make_rdma(d, c):
            return pltpu.make_async_remote_copy(
                src_ref=send_buf.at[d, c],
                dst_ref=recv_buf.at[d, c],
                send_sem=send_sems.at[d, c],
                recv_sem=recv_sems.at[d, c],
                device_id=(partners[d],),
                device_id_type=pl.DeviceIdType.MESH,
            )

        rdmas = [[make_rdma(d, c) for c in range(N_CHUNK)] for d in range(LOG2_N)]

        qb = lax.broadcasted_iota(jnp.int32, (SQ, SKV), 0) // 64
        kb = lax.broadcasted_iota(jnp.int32, (SQ, SKV), 1) // 64
        mask = (qb == kb) | (kb == 0) | ((qb + kb) % 3 == 0)
        bias = jnp.where(mask, 0.0, -1e9).astype(jnp.float32)
        bias4 = jnp.concatenate([bias] * HQ_SH, axis=0)

        wq = wq_ref[...].astype(jnp.bfloat16)
        wo = wo_ref[...].astype(jnp.bfloat16)
        for b in range(B):
            xb = x_ref[b].astype(jnp.bfloat16)
            q = jnp.dot(xb, wq, preferred_element_type=jnp.float32)
            s_list = []
            for h in range(HQ_SH):
                qh = q[:, h * DH:(h + 1) * DH].astype(jnp.bfloat16)
                kh = k_ref[b, :, h, :].astype(jnp.bfloat16)
                s_list.append(lax.dot_general(
                    qh, kh, (((1,), (1,)), ((), ())),
                    preferred_element_type=jnp.float32,
                ))
            s = jnp.concatenate(s_list, axis=0) * 0.125 + bias4
            m = jnp.max(s, axis=-1, keepdims=True)
            w = jnp.exp(s - m)
            w = (w / jnp.sum(w, axis=-1, keepdims=True)).astype(jnp.bfloat16)
            ctxs = []
            for h in range(HQ_SH):
                vh = v_ref[b, :, h, :].astype(jnp.bfloat16)
                ctxs.append(jnp.dot(w[h * SQ:(h + 1) * SQ, :], vh,
                                    preferred_element_type=jnp.float32))
            ctx = jnp.concatenate(ctxs, axis=1).astype(jnp.bfloat16)
            partial = jnp.dot(
                ctx, wo, preferred_element_type=jnp.float32
            ).astype(jnp.bfloat16)
            for j in range(2):
                c = 2 * b + j
                send_buf[0, c, :, :] = partial[:, j * D_HALF:(j + 1) * D_HALF]
                rdmas[0][c].start()

        for d in range(LOG2_N):
            for c in range(N_CHUNK):
                b, j = c // 2, c % 2
                rdmas[d][c].wait_recv()
                if d < LOG2_N - 1:
                    send_buf[d + 1, c, :, :] = send_buf[d, c] + recv_buf[d, c]
                    rdmas[d + 1][c].start()
                else:
                    out_ref[b, :, j * D_HALF:(j + 1) * D_HALF] = (
                        send_buf[d, c].astype(jnp.float32)
                        + recv_buf[d, c].astype(jnp.float32)
                    )

        for d in range(LOG2_N):
            for c in range(N_CHUNK):
                rdmas[d][c].wait_send()

    return pl.pallas_call(
        body,
        out_shape=jax.ShapeDtypeStruct((B, SQ, D_MODEL), jnp.float32),
        in_specs=[pl.BlockSpec(memory_space=pltpu.VMEM)] * 5,
        out_specs=pl.BlockSpec(memory_space=pltpu.VMEM),
        scratch_shapes=[
            pltpu.VMEM((LOG2_N, N_CHUNK, SQ, D_HALF), jnp.bfloat16),
            pltpu.VMEM((LOG2_N, N_CHUNK, SQ, D_HALF), jnp.bfloat16),
            pltpu.SemaphoreType.DMA((LOG2_N, N_CHUNK)),
            pltpu.SemaphoreType.DMA((LOG2_N, N_CHUNK)),
        ],
        compiler_params=pltpu.CompilerParams(collective_id=0),
    )(x, Wq, K_sh, V_sh, Wo)
